# baseline (speedup 1.0000x reference)
import functools

import jax
import jax.numpy as jnp
from jax import lax
from jax.experimental import pallas as pl
from jax.experimental.pallas import tpu as pltpu
from jax.experimental.pallas import tpu_sc as plsc

SEQ = 16384
DIM = 32
NC = 2
NS = 16
NW = NC * NS
TPW = SEQ // NW
CHUNK = 128
NCHUNK = TPW // CHUNK
NGRP = TPW // 16
SUB = 8


def _sc_body(seq_hbm, emb_hbm, sgn_hbm, out_hbm,
             idx_v, tile_v, sgn_v, part_v, sem_a, sem_b, ssem):
    wid = lax.axis_index("s") * NC + lax.axis_index("c")
    base = wid * TPW

    for j in range(NCHUNK):
        pltpu.sync_copy(seq_hbm.at[pl.ds(base + j * CHUNK, CHUNK)], idx_v.at[j])

    sgn_copies = [
        pltpu.make_async_copy(sgn_hbm.at[idx_v.at[j]],
                              sgn_v.at[pl.ds(j * CHUNK, CHUNK)], ssem)
        for j in range(NCHUNK)
    ]
    for c in sgn_copies:
        c.start()
    for c in sgn_copies:
        c.wait()

    def tanh_chunk(i, _):
        x = sgn_v[pl.ds(i * 16, 16)]
        e = jnp.exp(-2.0 * jnp.abs(x))
        sgn_v[pl.ds(i * 16, 16)] = jnp.sign(x) * (1.0 - e) / (1.0 + e)
        return 0

    lax.fori_loop(0, NGRP, tanh_chunk, 0)

    emb3 = emb_hbm.reshape(4, 8, emb_hbm.shape[-1])
    iota = lax.iota(jnp.int32, 16)
    dt_lo = iota // 8
    dt_hi = dt_lo + 2
    ds_sel = iota % 8

    def load_cvec(g):
        j = g // (CHUNK // 16)
        q = g % (CHUNK // 16)
        return idx_v[j, pl.ds(q * 16, 16)]

    def fire_sub(g, s, buf, bsem):
        c_vec = load_cvec(g)
        for k in range(SUB):
            c = c_vec[s * SUB + k]
            start = pl.multiple_of(lax.shift_right_logical(c, 7) * 128, 128)
            for dt in range(4):
                pltpu.make_async_copy(
                    emb3.at[dt, :, pl.ds(start, 128)],
                    tile_v.at[buf, k, dt], bsem).start()

    def drain_sub(buf, bsem):
        for _ in range(SUB * 4):
            pltpu.make_async_copy(
                emb3.at[0, :, pl.ds(0, 128)], tile_v.at[buf, 0, 0], bsem).wait()

    def extract_sub(g, s, buf, a_lo, a_hi):
        c_vec = load_cvec(g)
        w_vec = sgn_v[pl.ds(g * 16, 16)]
        bvec = jnp.full((16,), buf, jnp.int32)
        for k in range(SUB):
            rl = jnp.full((16,), c_vec[s * SUB + k] & 127, jnp.int32)
            tok = jnp.full((16,), k, jnp.int32)
            v_lo = plsc.load_gather(tile_v, [bvec, tok, dt_lo, ds_sel, rl])
            v_hi = plsc.load_gather(tile_v, [bvec, tok, dt_hi, ds_sel, rl])
            w = w_vec[s * SUB + k]
            a_lo = a_lo + w * v_lo
            a_hi = a_hi + w * v_hi
        return a_lo, a_hi

    def pair(i, carry):
        a_lo, a_hi = carry
        fire_sub(i, 1, 1, sem_b)
        drain_sub(0, sem_a)
        a_lo, a_hi = extract_sub(i, 0, 0, a_lo, a_hi)

        @pl.when(i + 1 < NGRP)
        def _():
            fire_sub(i + 1, 0, 0, sem_a)

        drain_sub(1, sem_b)
        a_lo, a_hi = extract_sub(i, 1, 1, a_lo, a_hi)
        return (a_lo, a_hi)

    z = jnp.zeros((16,), jnp.float32)
    fire_sub(0, 0, 0, sem_a)
    a_lo, a_hi = lax.fori_loop(0, NGRP, pair, (z, z))
    part_v[pl.ds(0, 16)] = a_lo
    part_v[pl.ds(16, 16)] = a_hi
    pltpu.sync_copy(part_v, out_hbm.at[wid])


def _reduce_body(p_ref, o_ref):
    o_ref[...] = jnp.sum(p_ref[...], axis=0, keepdims=True)


@jax.jit
def kernel(sequence, hash_embedding, sign_weight):
    seq = sequence.astype(jnp.int32)
    sc = pl.kernel(
        _sc_body,
        out_type=jax.ShapeDtypeStruct((NW, DIM), jnp.float32),
        mesh=plsc.VectorSubcoreMesh(core_axis_name="c", subcore_axis_name="s"),
        scratch_types=[
            pltpu.VMEM((NCHUNK, CHUNK), jnp.int32),
            pltpu.VMEM((2, SUB, 4, 8, 128), jnp.float32),
            pltpu.VMEM((TPW,), jnp.float32),
            pltpu.VMEM((DIM,), jnp.float32),
            pltpu.SemaphoreType.DMA,
            pltpu.SemaphoreType.DMA,
            pltpu.SemaphoreType.DMA,
        ],
        compiler_params=pltpu.CompilerParams(needs_layout_passes=False),
    )
    partials = sc(seq, hash_embedding.T, sign_weight)
    out = pl.pallas_call(
        _reduce_body,
        out_shape=jax.ShapeDtypeStruct((1, DIM), jnp.float32),
    )(partials)
    return out.reshape(DIM)
